# baseline (speedup 1.0000x reference)
import functools

import jax
import jax.numpy as jnp
from jax import lax
from jax.experimental import pallas as pl
from jax.experimental.pallas import tpu as pltpu
from jax.experimental.pallas import tpu_sc as plsc

_CHUNK = 128
_NBUF = 5


@functools.lru_cache(maxsize=None)
def _make_sc_kernel(n_rows: int, seq_len: int, d: int):
    info = plsc.get_sparse_core_info()
    nc, ns = info.num_cores, info.num_subcores
    nw = nc * ns
    rows_per_w = n_rows // nw
    n_chunks = rows_per_w // _CHUNK

    mesh = plsc.VectorSubcoreMesh(core_axis_name="c", subcore_axis_name="s")

    @functools.partial(
        pl.kernel,
        mesh=mesh,
        compiler_params=pltpu.CompilerParams(use_tc_tiling_on_sc=False),
        out_type=jax.ShapeDtypeStruct((n_rows, d), jnp.float32),
        scratch_types=[
            pltpu.VMEM((n_chunks, _CHUNK), jnp.int32),
            [pltpu.VMEM((_CHUNK, d), jnp.float32)] * _NBUF,
            [pltpu.SemaphoreType.DMA] * _NBUF,
            [pltpu.SemaphoreType.DMA] * _NBUF,
            [pltpu.SemaphoreType.DMA] * _NBUF,
        ],
    )
    def k(x_hbm, tok_hbm, pos2_hbm, out_hbm, idx_v, bufs, psems, gsems, ssems):
        wid = lax.axis_index("s") * nc + lax.axis_index("c")
        base = wid * rows_per_w
        pltpu.sync_copy(x_hbm.at[wid], idx_v)

        def pos_slice(g):
            return pos2_hbm.at[pl.ds(lax.rem(g * _CHUNK, seq_len), _CHUNK)]

        def out_slice(g):
            return out_hbm.at[pl.ds(base + g * _CHUNK, _CHUNK)]

        def prefill_start(g, slot):
            pltpu.async_copy(pos_slice(g), bufs[slot], psems[slot])

        def prefill_wait(g, slot):
            pltpu.make_async_copy(pos_slice(g), bufs[slot], psems[slot]).wait()

        def gather_start(g, slot):
            pltpu.async_copy(
                tok_hbm.at[idx_v.at[g]], bufs[slot], gsems[slot], add=True
            )

        def gather_wait(g, slot):
            pltpu.make_async_copy(
                tok_hbm.at[idx_v.at[g]], bufs[slot], gsems[slot]
            ).wait()

        prefill_start(0, 0)
        prefill_start(1, 1)
        prefill_wait(0, 0)
        gather_start(0, 0)

        def outer(oi, carry):
            for b in range(_NBUF):
                g = oi * _NBUF + b
                p2 = (b + 2) % _NBUF
                p1 = (b + 1) % _NBUF

                @pl.when(g + 2 < n_chunks)
                def _():
                    @pl.when(g - 3 >= 0)
                    def _():
                        pltpu.make_async_copy(
                            bufs[p2], out_slice(g - 3), ssems[p2]
                        ).wait()

                    prefill_start(g + 2, p2)

                @pl.when(g + 1 < n_chunks)
                def _():
                    prefill_wait(g + 1, p1)
                    gather_start(g + 1, p1)

                gather_wait(g, b)
                pltpu.async_copy(bufs[b], out_slice(g), ssems[b])
            return carry

        lax.fori_loop(0, n_chunks // _NBUF, outer, 0)

        for b in range(_NBUF):
            g = n_chunks - _NBUF + b
            pltpu.make_async_copy(bufs[b], out_slice(g), ssems[b]).wait()

    return k


def kernel(x, token_table, pos_table):
    b, s = x.shape
    d = token_table.shape[1]
    n_rows = b * s
    info = plsc.get_sparse_core_info()
    nw = info.num_cores * info.num_subcores
    n_chunks = n_rows // nw // _CHUNK
    x_idx = x.astype(jnp.int32).reshape(nw, n_chunks, _CHUNK)
    pos2 = jnp.concatenate([pos_table, pos_table], axis=0)
    out = _make_sc_kernel(n_rows, s, d)(x_idx, token_table, pos2)
    return out.reshape(b, s, d)
